# baseline (speedup 1.0000x reference)
import functools

import jax
import jax.numpy as jnp
from jax import lax
from jax.experimental import pallas as pl
from jax.experimental.pallas import tpu as pltpu, tpu_sc as plsc

N = 10000
E = 320000
D = 128
HID = 2 * D
HALF = D // 2

NC = 2
NS = 16
LANES = 16

EDGES_PER_TILE = E // NS
CHUNK = 128
MAINCH = 156
ROWS_PER_TILE = 624


def _bn_body(h_ref, g_ref, b_ref, x2_ref):
    h = h_ref[...]
    mean = jnp.mean(h, axis=0, keepdims=True)
    var = jnp.mean((h - mean) * (h - mean), axis=0, keepdims=True)
    x = (h - mean) * lax.rsqrt(var + 1e-5) * g_ref[...][None, :] + b_ref[...][None, :]
    x = jnp.maximum(x, 0.0)
    x2_ref[0:N, :] = x[:, :HALF]
    x2_ref[N:2 * N, :] = x[:, HALF:]


def _bn(h, bn_gamma, bn_beta):
    return pl.pallas_call(
        _bn_body,
        out_shape=jax.ShapeDtypeStruct((2 * N, HALF), jnp.float32),
    )(h, bn_gamma, bn_beta)


def _edge_body(x2, srcf, dstf, ea2, tvec, zeros, acc,
               accum, sidx, didx, aidx, sdix, xg, ea, vp, tv,
               gsem0, gsem1, isem0, isem1, ssem):
    c = lax.axis_index("c")
    s = lax.axis_index("s")
    row0 = pl.multiple_of(s * (2 * ROWS_PER_TILE), 8)

    hrow0 = pl.multiple_of(s * ROWS_PER_TILE, 8)
    pltpu.sync_copy(zeros.at[pl.ds(hrow0, ROWS_PER_TILE)],
                    accum.at[pl.ds(hrow0, ROWS_PER_TILE)])

    @pl.when(s == NS - 1)
    def _():
        t0 = NS * ROWS_PER_TILE
        pltpu.sync_copy(zeros.at[pl.ds(t0, N - t0)],
                        accum.at[pl.ds(t0, N - t0)])

    pltpu.sync_copy(tvec, tv)
    plsc.subcore_barrier()

    tval = tv[...]
    iota = lax.iota(jnp.int32, LANES)
    gsems = (gsem0, gsem1)
    isems = (isem0, isem1)
    nch = jnp.where(s < 2, MAINCH + 2, MAINCH)

    def e0_of(j):
        return pl.multiple_of(
            jnp.where(j < MAINCH,
                      s * (MAINCH * CHUNK) + j * CHUNK,
                      E - 512 + (2 * s + (j - MAINCH)) * CHUNK), 8)

    def idx_fire(j, b):
        e0 = e0_of(j)
        pltpu.async_copy(srcf.at[pl.ds(e0, CHUNK)], sidx.at[b], isems[b])
        pltpu.async_copy(dstf.at[pl.ds(e0, CHUNK)], didx.at[b], isems[b])

    def idx_drain(j, b):
        e0 = e0_of(j)
        pltpu.make_async_copy(srcf.at[pl.ds(e0, CHUNK)], sidx.at[b], isems[b]).wait()
        pltpu.make_async_copy(dstf.at[pl.ds(e0, CHUNK)], didx.at[b], isems[b]).wait()

    def gather_fire(j, b):
        e0 = e0_of(j)
        for k in range(CHUNK // LANES):
            sidx[b, pl.ds(k * LANES, LANES)] = (
                sidx[b, pl.ds(k * LANES, LANES)] + c * N)
            aidx[b, pl.ds(k * LANES, LANES)] = (iota + (e0 + k * LANES)) * 2 + c
        pltpu.async_copy(x2.at[sidx.at[b]], xg.at[b], gsems[b])
        pltpu.async_copy(ea2.at[aidx.at[b]], ea.at[b], gsems[b])

    def gather_drain(j, b):
        pltpu.make_async_copy(x2.at[sidx.at[b]], xg.at[b], gsems[b]).wait()
        pltpu.make_async_copy(ea2.at[aidx.at[b]], ea.at[b], gsems[b]).wait()

    UE = 8

    def compute(j, b):
        for k in range(CHUNK // LANES):
            sl = pl.ds(k * LANES, LANES)
            sdix[sl] = didx[b, sl]

        def blk(i, carry2):
            base = i * UE
            for k in range(UE):
                e = base + k
                for jj in range(HALF // LANES):
                    sl = pl.ds(jj * LANES, LANES)
                    m = jnp.maximum(xg[b, e, sl] + ea[b, e, sl], 0.0) + 1e-7
                    p = jnp.exp(tval * m)
                    vp[e, pl.ds(jj * LANES, LANES)] = p
                    vp[e, pl.ds(HALF + jj * LANES, LANES)] = p * m
            return carry2

        lax.fori_loop(0, CHUNK // UE, blk, 0)

    def scatter_fire(j):
        pltpu.async_copy(vp, accum.at[sdix], ssem, add=True)

    def scatter_drain(j):
        pltpu.make_async_copy(vp, accum.at[sdix], ssem).wait()

    idx_fire(0, 0)
    idx_fire(1, 1)
    idx_drain(0, 0)
    gather_fire(0, 0)

    def chunk_body(i, carry):
        j = i * 2

        def halfstep(j, b):
            gather_drain(j, b)

            @pl.when(j + 1 < nch)
            def _():
                idx_drain(j + 1, 1 - b)
                gather_fire(j + 1, 1 - b)

            @pl.when(j >= 1)
            def _():
                scatter_drain(j - 1)

            compute(j, b)
            scatter_fire(j)

            @pl.when(j + 2 < nch)
            def _():
                idx_fire(j + 2, b)

        halfstep(j, 0)
        halfstep(j + 1, 1)
        return carry

    lax.fori_loop(0, nch // 2, chunk_body, 0)
    scatter_drain(nch - 1)
    plsc.subcore_barrier()
    out0 = pl.multiple_of(c * N + hrow0, 8)
    pltpu.sync_copy(accum.at[pl.ds(hrow0, ROWS_PER_TILE)],
                    acc.at[pl.ds(out0, ROWS_PER_TILE)])

    @pl.when(s == NS - 1)
    def _():
        tail0 = NS * ROWS_PER_TILE
        ntail = N - tail0
        pltpu.sync_copy(accum.at[pl.ds(tail0, ntail)],
                        acc.at[pl.ds(pl.multiple_of(c * N + tail0, 8), ntail)])


def _edges(x2, srcf, dstf, ea2, tvec, zeros):
    mesh = plsc.VectorSubcoreMesh(core_axis_name="c", subcore_axis_name="s")
    f = functools.partial(
        pl.kernel,
        mesh=mesh,
        compiler_params=pltpu.CompilerParams(use_tc_tiling_on_sc=False),
        out_type=jax.ShapeDtypeStruct((2 * N, D), jnp.float32),
        scratch_types=[
            pltpu.VMEM_SHARED((N, D), jnp.float32),
            pltpu.VMEM((2, CHUNK), jnp.int32),
            pltpu.VMEM((2, CHUNK), jnp.int32),
            pltpu.VMEM((2, CHUNK), jnp.int32),
            pltpu.VMEM((CHUNK,), jnp.int32),
            pltpu.VMEM((2, CHUNK, HALF), jnp.float32),
            pltpu.VMEM((2, CHUNK, HALF), jnp.float32),
            pltpu.VMEM((CHUNK, D), jnp.float32),
            pltpu.VMEM((LANES,), jnp.float32),
            pltpu.SemaphoreType.DMA,
            pltpu.SemaphoreType.DMA,
            pltpu.SemaphoreType.DMA,
            pltpu.SemaphoreType.DMA,
            pltpu.SemaphoreType.DMA,
        ],
    )(_edge_body)
    return f(x2, srcf, dstf, ea2, tvec, zeros)


def _mlp_body(a_lo_ref, a_hi_ref, x_lo_ref, x_hi_ref, h_ref,
              w1_ref, b1_ref, lng_ref, lnb_ref, w2_ref, b2_ref, out_ref):
    a_lo = a_lo_ref[...]
    a_hi = a_hi_ref[...]
    num = jnp.concatenate([a_lo[:, HALF:], a_hi[:, HALF:]], axis=1)
    den = jnp.concatenate([a_lo[:, :HALF], a_hi[:, :HALF]], axis=1)
    x = jnp.concatenate([x_lo_ref[...], x_hi_ref[...]], axis=1)
    out = num / (den + 1e-16) + x
    out = jnp.dot(out, w1_ref[...], preferred_element_type=jnp.float32)
    out = out + b1_ref[...][None, :]
    mu = jnp.mean(out, axis=1, keepdims=True)
    sig = jnp.mean((out - mu) * (out - mu), axis=1, keepdims=True)
    out = (out - mu) * lax.rsqrt(sig + 1e-5) * lng_ref[...][None, :] + lnb_ref[...][None, :]
    out = jnp.maximum(out, 0.0)
    out = jnp.dot(out, w2_ref[...], preferred_element_type=jnp.float32)
    out_ref[...] = out + b2_ref[...][None, :] + h_ref[...]


def _mlp(acc, x2, h, W1, b1, ln_gamma, ln_beta, W2, b2):
    B = 1000
    grid = (N // B,)
    nb = N // B
    row_blk = lambda i: (i, 0)
    full1 = lambda w: pl.BlockSpec(w, lambda i: tuple(0 for _ in w))

    def off_blk(k):
        return pl.BlockSpec((B, HALF), lambda i, k=k: (i + k * nb, 0))

    return pl.pallas_call(
        _mlp_body,
        grid=grid,
        in_specs=[
            pl.BlockSpec((B, D), row_blk),
            pl.BlockSpec((B, D), lambda i: (i + nb, 0)),
            pl.BlockSpec((B, HALF), row_blk),
            off_blk(1),
            pl.BlockSpec((B, D), row_blk),
            full1((D, HID)),
            full1((HID,)),
            full1((HID,)),
            full1((HID,)),
            full1((HID, D)),
            full1((D,)),
        ],
        out_specs=pl.BlockSpec((B, D), row_blk),
        out_shape=jax.ShapeDtypeStruct((N, D), jnp.float32),
    )(acc, acc, x2, x2, h, W1, b1, ln_gamma, ln_beta, W2, b2)


def kernel(h, edge_index, edge_attr, bn_gamma, bn_beta, t,
           W1, b1, ln_gamma, ln_beta, W2, b2):
    x2 = _bn(h, bn_gamma, bn_beta)
    tvec = jnp.broadcast_to(t.astype(jnp.float32), (LANES,))
    zeros = jnp.zeros((N, D), jnp.float32)
    srcf = edge_index[0]
    dstf = edge_index[1]
    ea2 = edge_attr.reshape(2 * E, HALF)
    acc = _edges(x2, srcf, dstf, ea2, tvec, zeros)
    return _mlp(acc, x2, h, W1, b1, ln_gamma, ln_beta, W2, b2)


# speedup vs baseline: 4.2668x; 4.2668x over previous
import functools

import jax
import jax.numpy as jnp
from jax import lax
from jax.experimental import pallas as pl
from jax.experimental.pallas import tpu as pltpu, tpu_sc as plsc

N = 10000
E = 320000
D = 128
HID = 2 * D
HALF = D // 2

NC = 2
NS = 16
LANES = 16

EDGES_PER_TILE = E // NS
CHUNK = 128
MAINCH = 156
ROWS_PER_TILE = 624


def _bn_body(h_ref, g_ref, b_ref, x2_ref):
    h = h_ref[...]
    mean = jnp.mean(h, axis=0, keepdims=True)
    var = jnp.mean((h - mean) * (h - mean), axis=0, keepdims=True)
    x = (h - mean) * lax.rsqrt(var + 1e-5) * g_ref[...][None, :] + b_ref[...][None, :]
    x = jnp.maximum(x, 0.0)
    x2_ref[0:N, :] = x[:, :HALF]
    x2_ref[N:2 * N, :] = x[:, HALF:]


def _bn(h, bn_gamma, bn_beta):
    return pl.pallas_call(
        _bn_body,
        out_shape=jax.ShapeDtypeStruct((2 * N, HALF), jnp.float32),
    )(h, bn_gamma, bn_beta)


def _edge_body(x2, srcf, dstf, ea2, tvec, zeros, acc,
               accum, sidx, didx, aidx, sdix, sdix2, xg, ea, vp, vpm, tv,
               gsem0, gsem1, isem0, isem1, ssem):
    c = lax.axis_index("c")
    s = lax.axis_index("s")
    row0 = pl.multiple_of(s * (2 * ROWS_PER_TILE), 8)

    pltpu.sync_copy(zeros.at[pl.ds(row0, 2 * ROWS_PER_TILE)],
                    accum.at[pl.ds(row0, 2 * ROWS_PER_TILE)])

    @pl.when(s == NS - 1)
    def _():
        t0 = NS * 2 * ROWS_PER_TILE
        pltpu.sync_copy(zeros.at[pl.ds(t0, 2 * N - t0)],
                        accum.at[pl.ds(t0, 2 * N - t0)])

    pltpu.sync_copy(tvec, tv)
    plsc.subcore_barrier()

    tval = tv[...]
    iota = lax.iota(jnp.int32, LANES)
    gsems = (gsem0, gsem1)
    isems = (isem0, isem1)
    nch = jnp.where(s < 2, MAINCH + 2, MAINCH)

    def e0_of(j):
        return pl.multiple_of(
            jnp.where(j < MAINCH,
                      s * (MAINCH * CHUNK) + j * CHUNK,
                      E - 512 + (2 * s + (j - MAINCH)) * CHUNK), 8)

    def idx_fire(j, b):
        e0 = e0_of(j)
        pltpu.async_copy(srcf.at[pl.ds(e0, CHUNK)], sidx.at[b], isems[b])
        pltpu.async_copy(dstf.at[pl.ds(e0, CHUNK)], didx.at[b], isems[b])

    def idx_drain(j, b):
        e0 = e0_of(j)
        pltpu.make_async_copy(srcf.at[pl.ds(e0, CHUNK)], sidx.at[b], isems[b]).wait()
        pltpu.make_async_copy(dstf.at[pl.ds(e0, CHUNK)], didx.at[b], isems[b]).wait()

    def gather_fire(j, b):
        e0 = e0_of(j)
        for k in range(CHUNK // LANES):
            sidx[b, pl.ds(k * LANES, LANES)] = (
                sidx[b, pl.ds(k * LANES, LANES)] + c * N)
            aidx[b, pl.ds(k * LANES, LANES)] = (iota + (e0 + k * LANES)) * 2 + c
        pltpu.async_copy(x2.at[sidx.at[b]], xg.at[b], gsems[b])
        pltpu.async_copy(ea2.at[aidx.at[b]], ea.at[b], gsems[b])

    def gather_drain(j, b):
        pltpu.make_async_copy(x2.at[sidx.at[b]], xg.at[b], gsems[b]).wait()
        pltpu.make_async_copy(ea2.at[aidx.at[b]], ea.at[b], gsems[b]).wait()

    UE = 8

    def compute(j, b):
        for k in range(CHUNK // LANES):
            sl = pl.ds(k * LANES, LANES)
            d = didx[b, sl]
            sdix[sl] = d
            sdix2[sl] = d + N

        def blk(i, carry2):
            base = i * UE
            for k in range(UE):
                e = base + k
                for jj in range(HALF // LANES):
                    sl = pl.ds(jj * LANES, LANES)
                    m = jnp.maximum(xg[b, e, sl] + ea[b, e, sl], 0.0) + 1e-7
                    p = jnp.exp(tval * m)
                    vp[e, sl] = p
                    vpm[e, sl] = p * m
            return carry2

        lax.fori_loop(0, CHUNK // UE, blk, 0)

    def scatter_fire(j):
        pltpu.async_copy(vp, accum.at[sdix], ssem, add=True)
        pltpu.async_copy(vpm, accum.at[sdix2], ssem, add=True)

    def scatter_drain(j):
        pltpu.make_async_copy(vp, accum.at[sdix], ssem).wait()
        pltpu.make_async_copy(vpm, accum.at[sdix2], ssem).wait()

    idx_fire(0, 0)
    idx_fire(1, 1)
    idx_drain(0, 0)
    gather_fire(0, 0)

    def chunk_body(i, carry):
        j = i * 2

        def halfstep(j, b):
            gather_drain(j, b)

            @pl.when(j + 1 < nch)
            def _():
                idx_drain(j + 1, 1 - b)
                gather_fire(j + 1, 1 - b)

            @pl.when(j >= 1)
            def _():
                scatter_drain(j - 1)

            compute(j, b)
            scatter_fire(j)

            @pl.when(j + 2 < nch)
            def _():
                idx_fire(j + 2, b)

        halfstep(j, 0)
        halfstep(j + 1, 1)
        return carry

    lax.fori_loop(0, nch // 2, chunk_body, 0)
    scatter_drain(nch - 1)
    plsc.subcore_barrier()
    out0 = pl.multiple_of(c * (2 * N) + row0, 8)
    pltpu.sync_copy(accum.at[pl.ds(row0, 2 * ROWS_PER_TILE)],
                    acc.at[pl.ds(out0, 2 * ROWS_PER_TILE)])

    @pl.when(s == NS - 1)
    def _():
        tail0 = NS * 2 * ROWS_PER_TILE
        ntail = 2 * N - tail0
        pltpu.sync_copy(accum.at[pl.ds(tail0, ntail)],
                        acc.at[pl.ds(pl.multiple_of(c * (2 * N) + tail0, 8), ntail)])


def _edges(x2, srcf, dstf, ea2, tvec, zeros):
    mesh = plsc.VectorSubcoreMesh(core_axis_name="c", subcore_axis_name="s")
    f = functools.partial(
        pl.kernel,
        mesh=mesh,
        compiler_params=pltpu.CompilerParams(use_tc_tiling_on_sc=False),
        out_type=jax.ShapeDtypeStruct((4 * N, HALF), jnp.float32),
        scratch_types=[
            pltpu.VMEM_SHARED((2 * N, HALF), jnp.float32),
            pltpu.VMEM((2, CHUNK), jnp.int32),
            pltpu.VMEM((2, CHUNK), jnp.int32),
            pltpu.VMEM((2, CHUNK), jnp.int32),
            pltpu.VMEM((CHUNK,), jnp.int32),
            pltpu.VMEM((CHUNK,), jnp.int32),
            pltpu.VMEM((2, CHUNK, HALF), jnp.float32),
            pltpu.VMEM((2, CHUNK, HALF), jnp.float32),
            pltpu.VMEM((CHUNK, HALF), jnp.float32),
            pltpu.VMEM((CHUNK, HALF), jnp.float32),
            pltpu.VMEM((LANES,), jnp.float32),
            pltpu.SemaphoreType.DMA,
            pltpu.SemaphoreType.DMA,
            pltpu.SemaphoreType.DMA,
            pltpu.SemaphoreType.DMA,
            pltpu.SemaphoreType.DMA,
        ],
    )(_edge_body)
    return f(x2, srcf, dstf, ea2, tvec, zeros)


def _mlp_body(dl_ref, nl_ref, dh_ref, nh_ref, x_lo_ref, x_hi_ref, h_ref,
              w1_ref, b1_ref, lng_ref, lnb_ref, w2_ref, b2_ref, out_ref):
    num = jnp.concatenate([nl_ref[...], nh_ref[...]], axis=1)
    den = jnp.concatenate([dl_ref[...], dh_ref[...]], axis=1)
    x = jnp.concatenate([x_lo_ref[...], x_hi_ref[...]], axis=1)
    out = num / (den + 1e-16) + x
    out = jnp.dot(out, w1_ref[...], preferred_element_type=jnp.float32)
    out = out + b1_ref[...][None, :]
    mu = jnp.mean(out, axis=1, keepdims=True)
    sig = jnp.mean((out - mu) * (out - mu), axis=1, keepdims=True)
    out = (out - mu) * lax.rsqrt(sig + 1e-5) * lng_ref[...][None, :] + lnb_ref[...][None, :]
    out = jnp.maximum(out, 0.0)
    out = jnp.dot(out, w2_ref[...], preferred_element_type=jnp.float32)
    out_ref[...] = out + b2_ref[...][None, :] + h_ref[...]


def _mlp(acc, x2, h, W1, b1, ln_gamma, ln_beta, W2, b2):
    B = 1000
    grid = (N // B,)
    nb = N // B
    row_blk = lambda i: (i, 0)
    full1 = lambda w: pl.BlockSpec(w, lambda i: tuple(0 for _ in w))

    def off_blk(k):
        return pl.BlockSpec((B, HALF), lambda i, k=k: (i + k * nb, 0))

    return pl.pallas_call(
        _mlp_body,
        grid=grid,
        in_specs=[
            off_blk(0),
            off_blk(1),
            off_blk(2),
            off_blk(3),
            pl.BlockSpec((B, HALF), row_blk),
            off_blk(1),
            pl.BlockSpec((B, D), row_blk),
            full1((D, HID)),
            full1((HID,)),
            full1((HID,)),
            full1((HID,)),
            full1((HID, D)),
            full1((D,)),
        ],
        out_specs=pl.BlockSpec((B, D), row_blk),
        out_shape=jax.ShapeDtypeStruct((N, D), jnp.float32),
    )(acc, acc, acc, acc, x2, x2, h, W1, b1, ln_gamma, ln_beta, W2, b2)


def kernel(h, edge_index, edge_attr, bn_gamma, bn_beta, t,
           W1, b1, ln_gamma, ln_beta, W2, b2):
    x2 = _bn(h, bn_gamma, bn_beta)
    tvec = jnp.broadcast_to(t.astype(jnp.float32), (LANES,))
    zeros = jnp.zeros((2 * N, HALF), jnp.float32)
    srcf = edge_index[0]
    dstf = edge_index[1]
    ea2 = edge_attr.reshape(2 * E, HALF)
    acc = _edges(x2, srcf, dstf, ea2, tvec, zeros)
    return _mlp(acc, x2, h, W1, b1, ln_gamma, ln_beta, W2, b2)
